# Initial kernel scaffold; baseline (speedup 1.0000x reference)
import jax
import jax.numpy as jnp
from jax.experimental import pallas as pl


def kernel(inputs, gate_W, gate_b, expert_W, expert_b):
    raise NotImplementedError("write your pallas kernel here")


# speedup vs baseline: 1.1977x; 1.1977x over previous
import jax
import jax.numpy as jnp
from jax.experimental import pallas as pl
from jax.experimental.pallas import tpu as pltpu


def _moe_dense_body(x_ref, gwt_ref, gb_ref, ew_ref, eb_ref, o_ref):
    x = x_ref[...]
    logits = jax.lax.dot_general(
        x, gwt_ref[...], (((1,), (0,)), ((), ())),
        preferred_element_type=jnp.float32) + gb_ref[...]
    E = logits.shape[1]
    lane = jax.lax.broadcasted_iota(jnp.int32, logits.shape, 1)
    m1 = jnp.max(logits, axis=1, keepdims=True)
    idx1 = jnp.min(jnp.where(logits == m1, lane, E), axis=1, keepdims=True)
    oh1 = lane == idx1
    l2 = jnp.where(oh1, -1e30, logits)
    m2 = jnp.max(l2, axis=1, keepdims=True)
    idx2 = jnp.min(jnp.where(l2 == m2, lane, E), axis=1, keepdims=True)
    oh2 = lane == idx2
    e21 = jnp.exp(m2 - m1)
    w1 = 1.0 / (1.0 + e21)
    w2 = e21 / (1.0 + e21)
    wfull = jnp.where(oh1, w1, 0.0) + jnp.where(oh2, w2, 0.0)
    e = pl.program_id(1)
    wcol = jnp.sum(jnp.where(lane == e, wfull, 0.0), axis=1, keepdims=True)
    y = jax.lax.dot_general(
        x, ew_ref[0], (((1,), (1,)), ((), ())),
        preferred_element_type=jnp.float32)
    contrib = wcol * (y + eb_ref[0])

    @pl.when(e == 0)
    def _():
        o_ref[...] = contrib

    @pl.when(e != 0)
    def _():
        o_ref[...] += contrib


def kernel(inputs, gate_W, gate_b, expert_W, expert_b):
    T, D = inputs.shape
    E = expert_W.shape[0]
    TM = 512
    grid = (T // TM, E)
    gate_WT = gate_W.T
    gb = gate_b.reshape(1, E)
    eb3 = expert_b.reshape(E, 1, D)
    return pl.pallas_call(
        _moe_dense_body,
        grid=grid,
        in_specs=[
            pl.BlockSpec((TM, D), lambda m, e: (m, 0)),
            pl.BlockSpec((D, E), lambda m, e: (0, 0)),
            pl.BlockSpec((1, E), lambda m, e: (0, 0)),
            pl.BlockSpec((1, D, D), lambda m, e: (e, 0, 0)),
            pl.BlockSpec((1, 1, D), lambda m, e: (e, 0, 0)),
        ],
        out_specs=pl.BlockSpec((TM, D), lambda m, e: (m, 0)),
        out_shape=jax.ShapeDtypeStruct((T, D), inputs.dtype),
        compiler_params=pltpu.CompilerParams(
            dimension_semantics=("parallel", "arbitrary")),
    )(inputs, gate_WT, gb, expert_W, eb3)
